# Initial kernel scaffold; baseline (speedup 1.0000x reference)
import jax
import jax.numpy as jnp
from jax.experimental import pallas as pl


def kernel(x, adj, type_index, non_zero_index, non_zero_value, W_heads, a_heads, W_out, a_out, V_fm, W_attn, b_attn, h_attn, W_final, b_final):
    raise NotImplementedError("write your pallas kernel here")


# speedup vs baseline: 1.4489x; 1.4489x over previous
import functools

import jax
import jax.numpy as jnp
from jax.experimental import pallas as pl
from jax.experimental.pallas import tpu as pltpu

NFEAT = 128; NHID = 64; NCLASS = 40; NHEADS = 8; ALPHA = 0.2
N_NODES = 10000; N_EDGES = 320000; B = 4096; NNZ = 16


def _dense1_body(x_ref, w_ref, asrc_ref, adst_ref, h_ref, fs_ref, fd_ref):
    h = x_ref[...] @ w_ref[...]
    h_ref[...] = h
    fs_ref[...] = h @ asrc_ref[...]
    fd_ref[...] = h @ adst_ref[...]


def _dense1(x, w_cat, a_src_m, a_dst_m):
    n = x.shape[0]
    blk = 1000
    grid = (n // blk,)
    return pl.pallas_call(
        _dense1_body,
        grid=grid,
        in_specs=[
            pl.BlockSpec((blk, NFEAT), lambda i: (i, 0)),
            pl.BlockSpec((NFEAT, NHEADS * NHID), lambda i: (0, 0)),
            pl.BlockSpec((NHEADS * NHID, NHEADS), lambda i: (0, 0)),
            pl.BlockSpec((NHEADS * NHID, NHEADS), lambda i: (0, 0)),
        ],
        out_specs=[
            pl.BlockSpec((blk, NHEADS * NHID), lambda i: (i, 0)),
            pl.BlockSpec((blk, NHEADS), lambda i: (i, 0)),
            pl.BlockSpec((blk, NHEADS), lambda i: (i, 0)),
        ],
        out_shape=[
            jax.ShapeDtypeStruct((n, NHEADS * NHID), jnp.float32),
            jax.ShapeDtypeStruct((n, NHEADS), jnp.float32),
            jax.ShapeDtypeStruct((n, NHEADS), jnp.float32),
        ],
    )(x, w_cat, a_src_m, a_dst_m)


def kernel(x, adj, type_index, non_zero_index, non_zero_value, W_heads, a_heads,
           W_out, a_out, V_fm, W_attn, b_attn, h_attn, W_final, b_final):
    src = adj[0]
    dst = adj[1]
    N = x.shape[0]

    w_cat = jnp.transpose(W_heads, (1, 0, 2)).reshape(NFEAT, NHEADS * NHID)
    a2 = a_heads.reshape(NHEADS, 2 * NHID)
    a_src = a2[:, :NHID]
    a_dst = a2[:, NHID:]
    eye = jnp.eye(NHEADS, dtype=jnp.float32)
    a_src_m = (a_src[:, :, None] * eye[:, None, :]).reshape(NHEADS * NHID, NHEADS)
    a_dst_m = (a_dst[:, :, None] * eye[:, None, :]).reshape(NHEADS * NHID, NHEADS)

    h_all, f_src, f_dst = _dense1(x, w_cat, a_src_m, a_dst_m)

    e = f_src[src] + f_dst[dst]
    w = jnp.exp(-jnp.where(e >= 0, e, ALPHA * e))
    rowsum = jax.ops.segment_sum(w, src, num_segments=N)
    w_rep = jnp.repeat(w, NHID, axis=1)
    acc = jax.ops.segment_sum(w_rep * h_all[dst], src, num_segments=N)
    x1 = acc / (jnp.repeat(rowsum, NHID, axis=1) + 1e-16)
    x1 = jax.nn.elu(x1)

    a_out2 = a_out.reshape(2 * NHID)
    h2 = x1 @ W_out
    f2s = h2 @ a_out2[:NHID]
    f2d = h2 @ a_out2[NHID:]
    e2 = f2s[src] + f2d[dst]
    w2 = jnp.exp(-jnp.where(e2 >= 0, e2, ALPHA * e2))
    rowsum2 = jax.ops.segment_sum(w2, src, num_segments=N)
    acc2 = jax.ops.segment_sum(w2[:, None] * h2[dst], src, num_segments=N)
    gat = jax.nn.elu(acc2 / (rowsum2[:, None] + 1e-16))

    v = V_fm[non_zero_index] * non_zero_value[:, :, None]
    sum_sq = jnp.square(jnp.sum(v, axis=1))
    sq_sum = jnp.sum(jnp.square(v), axis=1)
    fm_feature = 0.5 * (sum_sq - sq_sum)

    g = gat[type_index]
    feats = jnp.stack([g, fm_feature], axis=1)
    attn = jnp.tanh(feats @ W_attn + b_attn) @ h_attn
    a_w = jax.nn.softmax(attn, axis=1)
    weighted = feats * a_w[:, :, None]
    x_all = weighted.reshape(weighted.shape[0], -1)
    x_all = x_all @ W_final + b_final
    return jax.nn.log_softmax(x_all, axis=1)


# speedup vs baseline: 2.9295x; 2.0219x over previous
import functools

import jax
import jax.numpy as jnp
from jax import lax
from jax.experimental import pallas as pl
from jax.experimental.pallas import tpu as pltpu
from jax.experimental.pallas import tpu_sc as plsc

NFEAT = 128; NHID = 64; NCLASS = 40; NHEADS = 8; ALPHA = 0.2
N_NODES = 10000; N_EDGES = 320000; B = 4096; NNZ = 16

N2 = 10240
NPT = N2 // 16
BLK = 1024
NPAIR = NHEADS // 2
PW = 2 * NHID


def _scale_rows2(w0_v, w1_v, rows_v, n_groups):
    def group_body(gg, carry):
        base = pl.multiple_of(gg * 16, 16)
        w0 = w0_v[pl.ds(base, 16)]
        w1 = w1_v[pl.ds(base, 16)]
        for j in range(16):
            wb0 = jnp.broadcast_to(w0[j], (16,))
            wb1 = jnp.broadcast_to(w1[j], (16,))
            for f in range(NHID // 16):
                fsl = pl.ds(f * 16, 16)
                rows_v[base + j, fsl] = rows_v[base + j, fsl] * wb0
                fsl2 = pl.ds(NHID + f * 16, 16)
                rows_v[base + j, fsl2] = rows_v[base + j, fsl2] * wb1
        return carry
    lax.fori_loop(0, n_groups, group_body, 0)


def _dense1_body(x_ref, w_ref, asrc_ref, adst_ref, h_ref, fs_ref, fd_ref):
    h = jnp.dot(x_ref[...], w_ref[0], preferred_element_type=jnp.float32)
    h_ref[0] = h
    fs_ref[0] = jnp.dot(h, asrc_ref[0], preferred_element_type=jnp.float32)
    fd_ref[0] = jnp.dot(h, adst_ref[0], preferred_element_type=jnp.float32)


def _dense1(x_p, wpair, asrc, adst):
    return pl.pallas_call(
        _dense1_body,
        grid=(N2 // BLK, NPAIR),
        in_specs=[
            pl.BlockSpec((BLK, NFEAT), lambda i, j: (i, 0)),
            pl.BlockSpec((1, NFEAT, PW), lambda i, j: (j, 0, 0)),
            pl.BlockSpec((1, PW, 2), lambda i, j: (j, 0, 0)),
            pl.BlockSpec((1, PW, 2), lambda i, j: (j, 0, 0)),
        ],
        out_specs=[
            pl.BlockSpec((1, BLK, PW), lambda i, j: (j, i, 0)),
            pl.BlockSpec((1, BLK, 2), lambda i, j: (j, i, 0)),
            pl.BlockSpec((1, BLK, 2), lambda i, j: (j, i, 0)),
        ],
        out_shape=[
            jax.ShapeDtypeStruct((NPAIR, N2, PW), jnp.float32),
            jax.ShapeDtypeStruct((NPAIR, N2, 2), jnp.float32),
            jax.ShapeDtypeStruct((NPAIR, N2, 2), jnp.float32),
        ],
    )(x_p, wpair, asrc, adst)


EPW = N_EDGES // 32
SEG = 1152
WSEG = 16 * SEG
CB = 128


def _binedges(src_e, dst_e):
    @functools.partial(
        pl.kernel,
        out_type=[
            jax.ShapeDtypeStruct((32 * WSEG + 32 * 128,), jnp.int32),
            jax.ShapeDtypeStruct((32 * WSEG + 32 * 128,), jnp.int32),
            jax.ShapeDtypeStruct((512,), jnp.int32),
        ],
        mesh=plsc.VectorSubcoreMesh(core_axis_name="c", subcore_axis_name="s"),
        scratch_types=[
            pltpu.VMEM((EPW + 112,), jnp.int32),
            pltpu.VMEM((EPW + 112,), jnp.int32),
            pltpu.VMEM((EPW // 128 + 1, 128), jnp.int32),
            pltpu.VMEM((EPW + 112,), jnp.int32),
            pltpu.VMEM((16, 128), jnp.int32),
            pltpu.VMEM((16, 128), jnp.int32),
            pltpu.VMEM((16, 128), jnp.int32),
            pltpu.VMEM((16,), jnp.int32),
        ],
    )
    def k(src_r, dst_r, bs_out, bd_out, cnt_out, srcb, dstb, aidx, abuf,
          sbuf_s, sbuf_d, sbuf_a, cbuf):
        c = lax.axis_index("c")
        s = lax.axis_index("s")
        wid = c * 16 + s
        gbase = wid * WSEG
        pltpu.sync_copy(src_r.at[pl.ds(wid * EPW, EPW)], srcb.at[pl.ds(0, EPW)])
        pltpu.sync_copy(dst_r.at[pl.ds(wid * EPW, EPW)], dstb.at[pl.ds(0, EPW)])
        iota = lax.iota(jnp.int32, 16)

        def vbody(i, offs):
            sl = pl.ds(pl.multiple_of(i * 16, 16), 16)
            srcv = srcb[sl]
            bkt = jnp.bitwise_and(srcv, 15)
            addr = jnp.zeros((16,), jnp.int32)
            for j in range(16):
                bj = bkt[j]
                aj = jnp.int32(0)
                for b in range(16):
                    aj = lax.select(bj == b, jnp.int32(b * SEG) + offs[b], aj)
                addr = addr + jnp.where(iota == j,
                                        jnp.broadcast_to(aj, (16,)), 0)
                offs = tuple(
                    lax.select(bj == b, offs[b] + 1, offs[b])
                    for b in range(16))
            abuf[sl] = addr + gbase
            return offs
        offs = lax.fori_loop(0, EPW // 16, vbody,
                             tuple(jnp.int32(0) for _ in range(16)))

        trash = jnp.broadcast_to(32 * WSEG + wid * 128, (16,)) + iota
        for j in range(7):
            abuf[pl.ds(EPW + j * 16, 16)] = trash + j * 16

        def cpbody(kk, carry):
            for j in range(8):
                aidx[kk, pl.ds(j * 16, 16)] = abuf[
                    pl.ds(pl.multiple_of(kk * 128 + j * 16, 16), 16)]
            return carry
        lax.fori_loop(0, EPW // 128 + 1, cpbody, 0)

        def sbody(kk, carry):
            pltpu.sync_copy(srcb.at[pl.ds(kk * 128, 128)],
                            bs_out.at[aidx.at[kk]])
            pltpu.sync_copy(dstb.at[pl.ds(kk * 128, 128)],
                            bd_out.at[aidx.at[kk]])
            return carry
        lax.fori_loop(0, EPW // 128 + 1, sbody, 0)

        rounded = jnp.zeros((16,), jnp.int32)
        for b in range(16):
            raw = offs[b]
            sent_s = jnp.broadcast_to(jnp.int32(N_NODES + b), (16,))
            sent_d = jnp.zeros((16,), jnp.int32)
            for j in range(CB // 16):
                sbuf_s[b, pl.ds(j * 16, 16)] = sent_s
                sbuf_d[b, pl.ds(j * 16, 16)] = sent_d
                pad_addr = (jnp.broadcast_to(gbase + b * SEG + raw + j * 16,
                                             (16,)) + iota)
                sbuf_a[b, pl.ds(j * 16, 16)] = pad_addr
            rb = jnp.right_shift(raw + CB - 1, 7) << 7
            rounded = jnp.where(iota == b, rb, rounded)

        def pbody(kk, carry):
            pltpu.sync_copy(sbuf_s.at[kk], bs_out.at[sbuf_a.at[kk]])
            pltpu.sync_copy(sbuf_d.at[kk], bd_out.at[sbuf_a.at[kk]])
            return carry
        lax.fori_loop(0, 16, pbody, 0)

        cbuf[pl.ds(0, 16)] = rounded
        pltpu.sync_copy(cbuf, cnt_out.at[pl.ds(wid * 16, 16)])

    return k(src_e, dst_e)


def _edges1(bsrc, bdst, cnts, fs_flat, fd_flat, h_flat, z128, z1):
    PPC = NPAIR // 2

    @functools.partial(
        pl.kernel,
        out_type=[
            jax.ShapeDtypeStruct((NPAIR * N2, PW), jnp.float32),
            jax.ShapeDtypeStruct((NPAIR * 2 * N2,), jnp.float32),
        ],
        mesh=plsc.VectorSubcoreMesh(core_axis_name="c", subcore_axis_name="s"),
        scratch_types=[
            pltpu.VMEM_SHARED((N2, PW), jnp.float32),
            pltpu.VMEM_SHARED((2 * N2,), jnp.float32),
            pltpu.VMEM((512,), jnp.int32),
            pltpu.VMEM((CB,), jnp.int32),
            pltpu.VMEM((CB,), jnp.int32),
            pltpu.VMEM((CB,), jnp.int32),
            pltpu.VMEM((CB,), jnp.int32),
            pltpu.VMEM((CB,), jnp.int32),
            pltpu.VMEM((CB,), jnp.float32),
            pltpu.VMEM((CB,), jnp.float32),
            pltpu.VMEM((CB,), jnp.float32),
            pltpu.VMEM((CB,), jnp.float32),
            pltpu.VMEM((CB,), jnp.int32),
            pltpu.VMEM((CB,), jnp.int32),
            pltpu.VMEM((CB, PW), jnp.float32),
        ],
    )
    def k(bs_r, bd_r, cnt_r, fs_r, fd_r, h_r, z128_r, z1_r, acc_out, rs_out,
          acc_sh, rs_sh, cbuf, sidx, didx, ridx, fidx, fidx2, fs_v, fd_v,
          w0_v, w1_v, rsi0, rsi1, rows_v):
        c = lax.axis_index("c")
        s = lax.axis_index("s")
        pltpu.sync_copy(cnt_r, cbuf)
        s16 = jnp.broadcast_to(s, (16,))
        iota = lax.iota(jnp.int32, 16)
        for pi in range(PPC):
            p = c * PPC + pi
            poff = p * N2
            pltpu.sync_copy(z128_r.at[pl.ds(s * NPT, NPT)],
                            acc_sh.at[pl.ds(s * NPT, NPT)])
            pltpu.sync_copy(z1_r.at[pl.ds(s * 2 * NPT, 2 * NPT)],
                            rs_sh.at[pl.ds(s * 2 * NPT, 2 * NPT)])
            plsc.subcore_barrier()

            def chunk_body(kk, seg_base):
                base = seg_base + kk * CB
                pltpu.sync_copy(bs_r.at[pl.ds(base, CB)], sidx)
                pltpu.sync_copy(bd_r.at[pl.ds(base, CB)], didx)
                for i in range(CB // 16):
                    sl = pl.ds(i * 16, 16)
                    ridx[sl] = didx[sl] + poff
                pltpu.sync_copy(h_r.at[ridx], rows_v)
                for q in range(2):
                    w_v = w0_v if q == 0 else w1_v
                    rsi = rsi0 if q == 0 else rsi1
                    for i in range(CB // 16):
                        sl = pl.ds(i * 16, 16)
                        fidx[sl] = sidx[sl] * 2 + (2 * poff + q)
                        fidx2[sl] = didx[sl] * 2 + (2 * poff + q)
                        rsi[sl] = sidx[sl] * 2 + q
                    pltpu.sync_copy(fs_r.at[fidx], fs_v)
                    pltpu.sync_copy(fd_r.at[fidx2], fd_v)
                    for i in range(CB // 16):
                        sl = pl.ds(i * 16, 16)
                        e = fs_v[sl] + fd_v[sl]
                        e = jnp.where(e >= 0.0, e, ALPHA * e)
                        w_v[sl] = jnp.exp(-e)
                pltpu.sync_copy(w0_v, rs_sh.at[rsi0], add=True)
                pltpu.sync_copy(w1_v, rs_sh.at[rsi1], add=True)
                _scale_rows2(w0_v, w1_v, rows_v, CB // 16)
                pltpu.sync_copy(rows_v, acc_sh.at[sidx], add=True)
                return seg_base

            def seg_body(w, carry):
                row = cbuf[pl.ds(pl.multiple_of(w * 16, 16), 16)]
                cnt = jnp.int32(0)
                for b in range(16):
                    cnt = lax.select(s == b, row[b], cnt)
                nch = jnp.right_shift(cnt, 7)
                seg_base = w * WSEG + s * SEG
                lax.fori_loop(0, nch, chunk_body, seg_base)
                return carry
            lax.fori_loop(0, 32, seg_body, 0)

            plsc.subcore_barrier()
            pltpu.sync_copy(acc_sh.at[pl.ds(s * NPT, NPT)],
                            acc_out.at[pl.ds(poff + s * NPT, NPT)])
            pltpu.sync_copy(rs_sh.at[pl.ds(s * 2 * NPT, 2 * NPT)],
                            rs_out.at[pl.ds(2 * poff + s * 2 * NPT, 2 * NPT)])

    return k(bsrc, bdst, cnts, fs_flat, fd_flat, h_flat, z128, z1)


def _dense2_body(acc_ref, rs_ref, wout_ref, aos_ref, aod_ref,
                 h2_ref, f2s_ref, f2d_ref):
    h2 = jnp.zeros((BLK, NHID), jnp.float32)
    for p in range(NPAIR):
        for q in range(2):
            x1h = (acc_ref[p, :, q * NHID:(q + 1) * NHID]
                   / (rs_ref[p, :, q:q + 1] + 1e-16))
            x1h = jnp.where(x1h > 0.0, x1h, jnp.exp(x1h) - 1.0)
            h2 = h2 + jnp.dot(x1h, wout_ref[2 * p + q],
                              preferred_element_type=jnp.float32)
    h2_ref[...] = jnp.concatenate([h2, jnp.zeros((BLK, NHID), jnp.float32)],
                                  axis=1)
    f2s_ref[...] = jnp.dot(h2, aos_ref[...], preferred_element_type=jnp.float32)
    f2d_ref[...] = jnp.dot(h2, aod_ref[...], preferred_element_type=jnp.float32)


def _dense2(acc4, rs4, wout3, aos, aod):
    return pl.pallas_call(
        _dense2_body,
        grid=(N2 // BLK,),
        in_specs=[
            pl.BlockSpec((NPAIR, BLK, PW), lambda i: (0, i, 0)),
            pl.BlockSpec((NPAIR, BLK, 2), lambda i: (0, i, 0)),
            pl.BlockSpec((NHEADS, NHID, NHID), lambda i: (0, 0, 0)),
            pl.BlockSpec((NHID, 1), lambda i: (0, 0)),
            pl.BlockSpec((NHID, 1), lambda i: (0, 0)),
        ],
        out_specs=[
            pl.BlockSpec((BLK, PW), lambda i: (i, 0)),
            pl.BlockSpec((BLK, 1), lambda i: (i, 0)),
            pl.BlockSpec((BLK, 1), lambda i: (i, 0)),
        ],
        out_shape=[
            jax.ShapeDtypeStruct((N2, PW), jnp.float32),
            jax.ShapeDtypeStruct((N2, 1), jnp.float32),
            jax.ShapeDtypeStruct((N2, 1), jnp.float32),
        ],
    )(acc4, rs4, wout3, aos, aod)


def _edges2(bsrc, bdst, cnts, f2s_flat, f2d_flat, h2, z128, z1):
    @functools.partial(
        pl.kernel,
        out_type=[
            jax.ShapeDtypeStruct((2 * N2, PW), jnp.float32),
            jax.ShapeDtypeStruct((2 * N2,), jnp.float32),
        ],
        mesh=plsc.VectorSubcoreMesh(core_axis_name="c", subcore_axis_name="s"),
        scratch_types=[
            pltpu.VMEM_SHARED((N2, PW), jnp.float32),
            pltpu.VMEM_SHARED((N2,), jnp.float32),
            pltpu.VMEM((512,), jnp.int32),
            pltpu.VMEM((CB,), jnp.int32),
            pltpu.VMEM((CB,), jnp.int32),
            pltpu.VMEM((CB,), jnp.float32),
            pltpu.VMEM((CB,), jnp.float32),
            pltpu.VMEM((CB,), jnp.float32),
            pltpu.VMEM((CB, PW), jnp.float32),
        ],
    )
    def k(bs_r, bd_r, cnt_r, fs_r, fd_r, h_r, z128_r, z1_r, acc_out, rs_out,
          acc_sh, rs_sh, cbuf, sidx, didx, fs_v, fd_v, w_v, rows_v):
        c = lax.axis_index("c")
        s = lax.axis_index("s")
        pltpu.sync_copy(cnt_r, cbuf)
        s16 = jnp.broadcast_to(s, (16,))
        iota = lax.iota(jnp.int32, 16)
        pltpu.sync_copy(z128_r.at[pl.ds(s * NPT, NPT)],
                        acc_sh.at[pl.ds(s * NPT, NPT)])
        pltpu.sync_copy(z1_r.at[pl.ds(s * NPT, NPT)],
                        rs_sh.at[pl.ds(s * NPT, NPT)])
        plsc.subcore_barrier()

        def chunk_body(kk, seg_base):
            base = seg_base + kk * CB
            pltpu.sync_copy(bs_r.at[pl.ds(base, CB)], sidx)
            pltpu.sync_copy(bd_r.at[pl.ds(base, CB)], didx)
            pltpu.sync_copy(h_r.at[didx], rows_v)
            pltpu.sync_copy(fs_r.at[sidx], fs_v)
            pltpu.sync_copy(fd_r.at[didx], fd_v)
            for i in range(CB // 16):
                sl = pl.ds(i * 16, 16)
                e = fs_v[sl] + fd_v[sl]
                e = jnp.where(e >= 0.0, e, ALPHA * e)
                w_v[sl] = jnp.exp(-e)
            pltpu.sync_copy(w_v, rs_sh.at[sidx], add=True)
            _scale_rows2(w_v, w_v, rows_v, CB // 16)
            pltpu.sync_copy(rows_v, acc_sh.at[sidx], add=True)
            return seg_base

        def seg_body(w, carry):
            w2 = c * 16 + w
            row = cbuf[pl.ds(pl.multiple_of(w2 * 16, 16), 16)]
            cnt = jnp.int32(0)
            for b in range(16):
                cnt = lax.select(s == b, row[b], cnt)
            nch = jnp.right_shift(cnt, 7)
            seg_base = w2 * WSEG + s * SEG
            lax.fori_loop(0, nch, chunk_body, seg_base)
            return carry
        lax.fori_loop(0, 16, seg_body, 0)

        plsc.subcore_barrier()
        pltpu.sync_copy(acc_sh.at[pl.ds(s * NPT, NPT)],
                        acc_out.at[pl.ds(c * N2 + s * NPT, NPT)])
        pltpu.sync_copy(rs_sh.at[pl.ds(s * NPT, NPT)],
                        rs_out.at[pl.ds(c * N2 + s * NPT, NPT)])

    return k(bsrc, bdst, cnts, f2s_flat, f2d_flat, h2, z128, z1)


def _merge_body(acc_ref, rs_ref, gat_ref):
    tot = acc_ref[0, :, :NHID] + acc_ref[1, :, :NHID]
    rs = rs_ref[0] + rs_ref[1] + 1e-16
    g = tot / rs
    g = jnp.where(g > 0.0, g, jnp.exp(g) - 1.0)
    gat_ref[...] = jnp.concatenate(
        [g, jnp.zeros((BLK, NHID), jnp.float32)], axis=1)


def _merge(acc2r, rs2r):
    return pl.pallas_call(
        _merge_body,
        grid=(N2 // BLK,),
        in_specs=[
            pl.BlockSpec((2, BLK, PW), lambda i: (0, i, 0)),
            pl.BlockSpec((2, BLK, 1), lambda i: (0, i, 0)),
        ],
        out_specs=pl.BlockSpec((BLK, PW), lambda i: (i, 0)),
        out_shape=jax.ShapeDtypeStruct((N2, PW), jnp.float32),
    )(acc2r, rs2r)


def _gather_fm(gat, type_index, nzi_flat, nzv_flat, zf):
    RPT = B // 32
    ZS = RPT * NFEAT
    HB = B // 2

    @functools.partial(
        pl.kernel,
        out_type=[
            jax.ShapeDtypeStruct((B, PW), jnp.float32),
            jax.ShapeDtypeStruct((B * NFEAT,), jnp.float32),
            jax.ShapeDtypeStruct((B * NFEAT,), jnp.float32),
        ],
        mesh=plsc.VectorSubcoreMesh(core_axis_name="c", subcore_axis_name="s"),
        scratch_types=[
            pltpu.VMEM_SHARED((HB * NFEAT,), jnp.float32),
            pltpu.VMEM_SHARED((HB * NFEAT,), jnp.float32),
            pltpu.VMEM((RPT,), jnp.int32),
            pltpu.VMEM((RPT, PW), jnp.float32),
            pltpu.VMEM((RPT * NNZ,), jnp.int32),
            pltpu.VMEM((RPT * NNZ,), jnp.float32),
            pltpu.VMEM((RPT * NNZ,), jnp.float32),
            pltpu.VMEM((RPT * NNZ,), jnp.int32),
        ],
    )
    def k(gat_r, ti_r, nzi_r, nzv_r, zf_r, g_out, s_out, s2_out,
          s_sh, s2_sh, tib, grows, nzib, nzvb, nzv2b, tidxb):
        c = lax.axis_index("c")
        s = lax.axis_index("s")
        row0 = c * HB + s * RPT
        pltpu.sync_copy(zf_r.at[pl.ds(s * ZS, ZS)], s_sh.at[pl.ds(s * ZS, ZS)])
        pltpu.sync_copy(zf_r.at[pl.ds(s * ZS, ZS)], s2_sh.at[pl.ds(s * ZS, ZS)])
        pltpu.sync_copy(ti_r.at[pl.ds(row0, RPT)], tib)
        pltpu.sync_copy(gat_r.at[tib], grows)
        pltpu.sync_copy(grows, g_out.at[pl.ds(row0, RPT)])
        pltpu.sync_copy(nzi_r.at[pl.ds(row0 * NNZ, RPT * NNZ)], nzib)
        pltpu.sync_copy(nzv_r.at[pl.ds(row0 * NNZ, RPT * NNZ)], nzvb)
        for r in range(RPT):
            sl = pl.ds(r * NNZ, 16)
            tidxb[sl] = nzib[sl] + (s * RPT + r) * NFEAT
            v = nzvb[sl]
            nzv2b[sl] = v * v
        plsc.subcore_barrier()
        pltpu.sync_copy(nzvb, s_sh.at[tidxb], add=True)
        pltpu.sync_copy(nzv2b, s2_sh.at[tidxb], add=True)
        plsc.subcore_barrier()
        pltpu.sync_copy(s_sh.at[pl.ds(s * ZS, ZS)],
                        s_out.at[pl.ds(c * HB * NFEAT + s * ZS, ZS)])
        pltpu.sync_copy(s2_sh.at[pl.ds(s * ZS, ZS)],
                        s2_out.at[pl.ds(c * HB * NFEAT + s * ZS, ZS)])

    return k(gat, type_index, nzi_flat, nzv_flat, zf)


def _fusion_body(g_ref, s_ref, s2_ref, vfm_ref, wa_ref, ba_ref, ha_ref,
                 wf_ref, bf_ref, out_ref):
    g = g_ref[:, :NHID]
    vfm = vfm_ref[...]
    sv = jnp.dot(s_ref[...], vfm, preferred_element_type=jnp.float32)
    s2v = jnp.dot(s2_ref[...], vfm * vfm, preferred_element_type=jnp.float32)
    fm = 0.5 * (sv * sv - s2v)
    ba = ba_ref[...]
    ha = ha_ref[...]
    wa = wa_ref[...]
    ag = jnp.dot(jnp.tanh(jnp.dot(g, wa, preferred_element_type=jnp.float32) + ba),
                 ha, preferred_element_type=jnp.float32)
    af = jnp.dot(jnp.tanh(jnp.dot(fm, wa, preferred_element_type=jnp.float32) + ba),
                 ha, preferred_element_type=jnp.float32)
    m = jnp.maximum(ag, af)
    e0 = jnp.exp(ag - m)
    e1 = jnp.exp(af - m)
    tot = e0 + e1
    a0 = e0 / tot
    a1 = e1 / tot
    wf = wf_ref[...]
    out = (jnp.dot(g * a0, wf[:NHID], preferred_element_type=jnp.float32)
           + jnp.dot(fm * a1, wf[NHID:], preferred_element_type=jnp.float32)
           + bf_ref[...])
    mx = jnp.max(out, axis=1, keepdims=True)
    sh = out - mx
    lse = jnp.log(jnp.sum(jnp.exp(sh), axis=1, keepdims=True))
    out_ref[...] = sh - lse


def _fusion(g, s, s2, vfm, wa, ba, ha, wf, bf):
    FB = 512
    return pl.pallas_call(
        _fusion_body,
        grid=(B // FB,),
        in_specs=[
            pl.BlockSpec((FB, PW), lambda i: (i, 0)),
            pl.BlockSpec((FB, NFEAT), lambda i: (i, 0)),
            pl.BlockSpec((FB, NFEAT), lambda i: (i, 0)),
            pl.BlockSpec((NFEAT, NHID), lambda i: (0, 0)),
            pl.BlockSpec((NHID, NHID), lambda i: (0, 0)),
            pl.BlockSpec((1, NHID), lambda i: (0, 0)),
            pl.BlockSpec((NHID, 1), lambda i: (0, 0)),
            pl.BlockSpec((2 * NHID, NCLASS), lambda i: (0, 0)),
            pl.BlockSpec((1, NCLASS), lambda i: (0, 0)),
        ],
        out_specs=pl.BlockSpec((FB, NCLASS), lambda i: (i, 0)),
        out_shape=jax.ShapeDtypeStruct((B, NCLASS), jnp.float32),
    )(g, s, s2, vfm, wa, ba, ha, wf, bf)


_SC_B = True
_SC_D = True
_SC_F = True


def _edges1_xla(src, dst, h3, fs3, fd3):
    accs, rss = [], []
    for p in range(NPAIR):
        e = fs3[p][src] + fd3[p][dst]
        w = jnp.exp(-jnp.where(e >= 0, e, ALPHA * e))
        rs_p = jax.ops.segment_sum(w, src, num_segments=N2)
        hd = h3[p][dst]
        w2 = jnp.repeat(w, NHID, axis=1)
        acc_p = jax.ops.segment_sum(w2 * hd, src, num_segments=N2)
        accs.append(acc_p)
        rss.append(rs_p)
    return jnp.stack(accs), jnp.stack(rss)


def _edges2_xla(src, dst, f2s, f2d, h2):
    e = f2s.reshape(N2)[src] + f2d.reshape(N2)[dst]
    w = jnp.exp(-jnp.where(e >= 0, e, ALPHA * e))
    rs = jax.ops.segment_sum(w, src, num_segments=N2)
    acc = jax.ops.segment_sum(w[:, None] * h2[dst], src, num_segments=N2)
    z = jnp.zeros_like(acc)
    return (jnp.stack([acc, z]), jnp.stack([rs, jnp.zeros_like(rs)])[..., None])


def _gather_fm_xla(gat, type_index, nzi, nzv):
    g = gat[type_index]
    rows = jnp.arange(B)[:, None]
    s = jnp.zeros((B, NFEAT), jnp.float32).at[rows, nzi].add(nzv)
    s2 = jnp.zeros((B, NFEAT), jnp.float32).at[rows, nzi].add(nzv * nzv)
    return g, s, s2


def kernel(x, adj, type_index, non_zero_index, non_zero_value, W_heads, a_heads,
           W_out, a_out, V_fm, W_attn, b_attn, h_attn, W_final, b_final):
    x_p = jnp.zeros((N2, NFEAT), jnp.float32).at[:N_NODES].set(x)
    wpair = (W_heads.reshape(NPAIR, 2, NFEAT, NHID)
             .transpose(0, 2, 1, 3).reshape(NPAIR, NFEAT, PW))
    a2 = a_heads.reshape(NHEADS, 2 * NHID)
    asrc = a2[:, :NHID].reshape(NPAIR, 2, NHID)
    adst = a2[:, NHID:].reshape(NPAIR, 2, NHID)
    eye2 = jnp.eye(2, dtype=jnp.float32)
    asrc_bd = (asrc[:, :, :, None] * eye2[None, :, None, :]).reshape(NPAIR, PW, 2)
    adst_bd = (adst[:, :, :, None] * eye2[None, :, None, :]).reshape(NPAIR, PW, 2)

    h3, fs3, fd3 = _dense1(x_p, wpair, asrc_bd, adst_bd)
    h_flat = h3.reshape(NPAIR * N2, PW)
    fs_flat = fs3.reshape(NPAIR * N2 * 2)
    fd_flat = fd3.reshape(NPAIR * N2 * 2)

    zbig = jnp.zeros((N2 * PW,), jnp.float32)
    z128 = zbig.reshape(N2, PW)
    z1 = zbig[:2 * N2]
    zf = zbig[:(B // 2) * NFEAT]

    src_e = adj[0]
    dst_e = adj[1]
    if _SC_B or _SC_D:
        bsrc, bdst, cnts = _binedges(src_e, dst_e)
    if _SC_B:
        acc_flat, rs_flat = _edges1(bsrc, bdst, cnts, fs_flat, fd_flat,
                                    h_flat, z128, z1)
        acc4 = acc_flat.reshape(NPAIR, N2, PW)
        rs4 = rs_flat.reshape(NPAIR, N2, 2)
    else:
        acc4, rs4 = _edges1_xla(src_e, dst_e, h3, fs3, fd3)

    wout3 = W_out.reshape(NHEADS, NHID, NHID)
    ao = a_out.reshape(2 * NHID)
    aos = ao[:NHID].reshape(NHID, 1)
    aod = ao[NHID:].reshape(NHID, 1)
    h2, f2s, f2d = _dense2(acc4, rs4, wout3, aos, aod)

    if _SC_D:
        acc2_flat, rs2_flat = _edges2(bsrc, bdst, cnts, f2s.reshape(N2),
                                      f2d.reshape(N2), h2, z128, z1[:N2])
        acc2r = acc2_flat.reshape(2, N2, PW)
        rs2r = rs2_flat.reshape(2, N2, 1)
    else:
        acc2r, rs2r = _edges2_xla(src_e, dst_e, f2s, f2d, h2)
    gat = _merge(acc2r, rs2r)

    if _SC_F:
        g, s_flat, s2_flat = _gather_fm(gat, type_index,
                                        non_zero_index.reshape(B * NNZ),
                                        non_zero_value.reshape(B * NNZ), zf)
        s = s_flat.reshape(B, NFEAT)
        s2 = s2_flat.reshape(B, NFEAT)
    else:
        g, s, s2 = _gather_fm_xla(gat, type_index, non_zero_index,
                                  non_zero_value)

    return _fusion(g, s, s2,
                   V_fm, W_attn, b_attn.reshape(1, NHID),
                   h_attn.reshape(NHID, 1), W_final,
                   b_final.reshape(1, NCLASS))


# speedup vs baseline: 3.1579x; 1.0780x over previous
import functools

import jax
import jax.numpy as jnp
from jax import lax
from jax.experimental import pallas as pl
from jax.experimental.pallas import tpu as pltpu
from jax.experimental.pallas import tpu_sc as plsc

NFEAT = 128; NHID = 64; NCLASS = 40; NHEADS = 8; ALPHA = 0.2
N_NODES = 10000; N_EDGES = 320000; B = 4096; NNZ = 16

N2 = 10240
NPT = N2 // 16
BLK = 1024
NPAIR = NHEADS // 2
PW = 2 * NHID


def _scale_rows2(w0_v, w1_v, rows_v, n_groups):
    def group_body(gg, carry):
        base = pl.multiple_of(gg * 16, 16)
        w0 = w0_v[pl.ds(base, 16)]
        w1 = w1_v[pl.ds(base, 16)]
        for j in range(16):
            wb0 = jnp.broadcast_to(w0[j], (16,))
            wb1 = jnp.broadcast_to(w1[j], (16,))
            for f in range(NHID // 16):
                fsl = pl.ds(f * 16, 16)
                rows_v[base + j, fsl] = rows_v[base + j, fsl] * wb0
                fsl2 = pl.ds(NHID + f * 16, 16)
                rows_v[base + j, fsl2] = rows_v[base + j, fsl2] * wb1
        return carry
    lax.fori_loop(0, n_groups, group_body, 0)


def _dense1_body(x_ref, w_ref, asrc_ref, adst_ref, h_ref, fs_ref, fd_ref):
    h = jnp.dot(x_ref[...], w_ref[0], preferred_element_type=jnp.float32)
    h_ref[0] = h
    fs_ref[0] = jnp.dot(h, asrc_ref[0], preferred_element_type=jnp.float32)
    fd_ref[0] = jnp.dot(h, adst_ref[0], preferred_element_type=jnp.float32)


def _dense1(x_p, wpair, asrc, adst):
    return pl.pallas_call(
        _dense1_body,
        grid=(N2 // BLK, NPAIR),
        in_specs=[
            pl.BlockSpec((BLK, NFEAT), lambda i, j: (i, 0)),
            pl.BlockSpec((1, NFEAT, PW), lambda i, j: (j, 0, 0)),
            pl.BlockSpec((1, PW, 2), lambda i, j: (j, 0, 0)),
            pl.BlockSpec((1, PW, 2), lambda i, j: (j, 0, 0)),
        ],
        out_specs=[
            pl.BlockSpec((1, BLK, PW), lambda i, j: (j, i, 0)),
            pl.BlockSpec((1, BLK, 2), lambda i, j: (j, i, 0)),
            pl.BlockSpec((1, BLK, 2), lambda i, j: (j, i, 0)),
        ],
        out_shape=[
            jax.ShapeDtypeStruct((NPAIR, N2, PW), jnp.float32),
            jax.ShapeDtypeStruct((NPAIR, N2, 2), jnp.float32),
            jax.ShapeDtypeStruct((NPAIR, N2, 2), jnp.float32),
        ],
    )(x_p, wpair, asrc, adst)


EPW = N_EDGES // 32
SEG = 1152
WSEG = 16 * SEG
CB = 128


def _binedges(src_e, dst_e):
    @functools.partial(
        pl.kernel,
        out_type=[
            jax.ShapeDtypeStruct((32 * WSEG + 32 * 128,), jnp.int32),
            jax.ShapeDtypeStruct((32 * WSEG + 32 * 128,), jnp.int32),
            jax.ShapeDtypeStruct((512,), jnp.int32),
        ],
        mesh=plsc.VectorSubcoreMesh(core_axis_name="c", subcore_axis_name="s"),
        scratch_types=[
            pltpu.VMEM((EPW + 112,), jnp.int32),
            pltpu.VMEM((EPW + 112,), jnp.int32),
            pltpu.VMEM((EPW // 128 + 1, 128), jnp.int32),
            pltpu.VMEM((EPW + 112,), jnp.int32),
            pltpu.VMEM((16, 128), jnp.int32),
            pltpu.VMEM((16, 128), jnp.int32),
            pltpu.VMEM((16, 128), jnp.int32),
            pltpu.VMEM((16,), jnp.int32),
        ],
    )
    def k(src_r, dst_r, bs_out, bd_out, cnt_out, srcb, dstb, aidx, abuf,
          sbuf_s, sbuf_d, sbuf_a, cbuf):
        c = lax.axis_index("c")
        s = lax.axis_index("s")
        wid = c * 16 + s
        gbase = wid * WSEG
        pltpu.sync_copy(src_r.at[pl.ds(wid * EPW, EPW)], srcb.at[pl.ds(0, EPW)])
        pltpu.sync_copy(dst_r.at[pl.ds(wid * EPW, EPW)], dstb.at[pl.ds(0, EPW)])
        iota = lax.iota(jnp.int32, 16)

        def vbody(i, offs):
            sl = pl.ds(pl.multiple_of(i * 16, 16), 16)
            srcv = srcb[sl]
            bkt = jnp.bitwise_and(srcv, 15)
            addr = jnp.zeros((16,), jnp.int32)
            for j in range(16):
                bj = bkt[j]
                aj = jnp.int32(0)
                for b in range(16):
                    aj = lax.select(bj == b, jnp.int32(b * SEG) + offs[b], aj)
                addr = addr + jnp.where(iota == j,
                                        jnp.broadcast_to(aj, (16,)), 0)
                offs = tuple(
                    lax.select(bj == b, offs[b] + 1, offs[b])
                    for b in range(16))
            abuf[sl] = addr + gbase
            return offs
        offs = lax.fori_loop(0, EPW // 16, vbody,
                             tuple(jnp.int32(0) for _ in range(16)))

        trash = jnp.broadcast_to(32 * WSEG + wid * 128, (16,)) + iota
        for j in range(7):
            abuf[pl.ds(EPW + j * 16, 16)] = trash + j * 16

        def cpbody(kk, carry):
            for j in range(8):
                aidx[kk, pl.ds(j * 16, 16)] = abuf[
                    pl.ds(pl.multiple_of(kk * 128 + j * 16, 16), 16)]
            return carry
        lax.fori_loop(0, EPW // 128 + 1, cpbody, 0)

        def sbody(kk, carry):
            pltpu.sync_copy(srcb.at[pl.ds(kk * 128, 128)],
                            bs_out.at[aidx.at[kk]])
            pltpu.sync_copy(dstb.at[pl.ds(kk * 128, 128)],
                            bd_out.at[aidx.at[kk]])
            return carry
        lax.fori_loop(0, EPW // 128 + 1, sbody, 0)

        rounded = jnp.zeros((16,), jnp.int32)
        for b in range(16):
            raw = offs[b]
            sent_s = jnp.broadcast_to(jnp.int32(N_NODES + b), (16,))
            sent_d = jnp.zeros((16,), jnp.int32)
            for j in range(CB // 16):
                sbuf_s[b, pl.ds(j * 16, 16)] = sent_s
                sbuf_d[b, pl.ds(j * 16, 16)] = sent_d
                pad_addr = (jnp.broadcast_to(gbase + b * SEG + raw + j * 16,
                                             (16,)) + iota)
                sbuf_a[b, pl.ds(j * 16, 16)] = pad_addr
            rb = jnp.right_shift(raw + CB - 1, 7) << 7
            rounded = jnp.where(iota == b, rb, rounded)

        def pbody(kk, carry):
            pltpu.sync_copy(sbuf_s.at[kk], bs_out.at[sbuf_a.at[kk]])
            pltpu.sync_copy(sbuf_d.at[kk], bd_out.at[sbuf_a.at[kk]])
            return carry
        lax.fori_loop(0, 16, pbody, 0)

        cbuf[pl.ds(0, 16)] = rounded
        pltpu.sync_copy(cbuf, cnt_out.at[pl.ds(wid * 16, 16)])

    return k(src_e, dst_e)


def _edges1(bsrc, bdst, cnts, fs_flat, fd_flat, h_flat, z128, z1):
    PPC = NPAIR // 2

    @functools.partial(
        pl.kernel,
        out_type=[
            jax.ShapeDtypeStruct((NPAIR * N2, PW), jnp.float32),
            jax.ShapeDtypeStruct((NPAIR * 2 * N2,), jnp.float32),
        ],
        mesh=plsc.VectorSubcoreMesh(core_axis_name="c", subcore_axis_name="s"),
        scratch_types=[
            pltpu.VMEM_SHARED((N2, PW), jnp.float32),
            pltpu.VMEM_SHARED((2 * N2,), jnp.float32),
            pltpu.VMEM((512,), jnp.int32),
            pltpu.VMEM((CB,), jnp.int32),
            pltpu.VMEM((CB,), jnp.int32),
            pltpu.VMEM((CB,), jnp.int32),
            pltpu.VMEM((CB,), jnp.int32),
            pltpu.VMEM((CB,), jnp.int32),
            pltpu.VMEM((CB,), jnp.float32),
            pltpu.VMEM((CB,), jnp.float32),
            pltpu.VMEM((CB,), jnp.float32),
            pltpu.VMEM((CB,), jnp.float32),
            pltpu.VMEM((CB,), jnp.int32),
            pltpu.VMEM((CB,), jnp.int32),
            pltpu.VMEM((CB, PW), jnp.float32),
            pltpu.SemaphoreType.DMA,
        ],
    )
    def k(bs_r, bd_r, cnt_r, fs_r, fd_r, h_r, z128_r, z1_r, acc_out, rs_out,
          acc_sh, rs_sh, cbuf, sidx, didx, ridx, fidx, fidx2, fs_v, fd_v,
          w0_v, w1_v, rsi0, rsi1, rows_v, sem):
        c = lax.axis_index("c")
        s = lax.axis_index("s")
        pltpu.sync_copy(cnt_r, cbuf)
        s16 = jnp.broadcast_to(s, (16,))
        iota = lax.iota(jnp.int32, 16)
        for pi in range(PPC):
            p = c * PPC + pi
            poff = p * N2
            pltpu.sync_copy(z128_r.at[pl.ds(s * NPT, NPT)],
                            acc_sh.at[pl.ds(s * NPT, NPT)])
            pltpu.sync_copy(z1_r.at[pl.ds(s * 2 * NPT, 2 * NPT)],
                            rs_sh.at[pl.ds(s * 2 * NPT, 2 * NPT)])
            plsc.subcore_barrier()

            def chunk_body(kk, seg_base):
                base = seg_base + kk * CB
                pltpu.sync_copy(bs_r.at[pl.ds(base, CB)], sidx)
                pltpu.sync_copy(bd_r.at[pl.ds(base, CB)], didx)
                for i in range(CB // 16):
                    sl = pl.ds(i * 16, 16)
                    ridx[sl] = didx[sl] + poff
                rows_cp = pltpu.async_copy(h_r.at[ridx], rows_v, sem)
                for q in range(2):
                    w_v = w0_v if q == 0 else w1_v
                    rsi = rsi0 if q == 0 else rsi1
                    for i in range(CB // 16):
                        sl = pl.ds(i * 16, 16)
                        fidx[sl] = sidx[sl] * 2 + (2 * poff + q)
                        fidx2[sl] = didx[sl] * 2 + (2 * poff + q)
                        rsi[sl] = sidx[sl] * 2 + q
                    pltpu.sync_copy(fs_r.at[fidx], fs_v)
                    pltpu.sync_copy(fd_r.at[fidx2], fd_v)
                    for i in range(CB // 16):
                        sl = pl.ds(i * 16, 16)
                        e = fs_v[sl] + fd_v[sl]
                        e = jnp.where(e >= 0.0, e, ALPHA * e)
                        w_v[sl] = jnp.exp(-e)
                pltpu.sync_copy(w0_v, rs_sh.at[rsi0], add=True)
                pltpu.sync_copy(w1_v, rs_sh.at[rsi1], add=True)
                rows_cp.wait()
                _scale_rows2(w0_v, w1_v, rows_v, CB // 16)
                pltpu.sync_copy(rows_v, acc_sh.at[sidx], add=True)
                return seg_base

            def seg_body(w, carry):
                row = cbuf[pl.ds(pl.multiple_of(w * 16, 16), 16)]
                cnt = jnp.int32(0)
                for b in range(16):
                    cnt = lax.select(s == b, row[b], cnt)
                nch = jnp.right_shift(cnt, 7)
                seg_base = w * WSEG + s * SEG
                lax.fori_loop(0, nch, chunk_body, seg_base)
                return carry
            lax.fori_loop(0, 32, seg_body, 0)

            plsc.subcore_barrier()
            pltpu.sync_copy(acc_sh.at[pl.ds(s * NPT, NPT)],
                            acc_out.at[pl.ds(poff + s * NPT, NPT)])
            pltpu.sync_copy(rs_sh.at[pl.ds(s * 2 * NPT, 2 * NPT)],
                            rs_out.at[pl.ds(2 * poff + s * 2 * NPT, 2 * NPT)])

    return k(bsrc, bdst, cnts, fs_flat, fd_flat, h_flat, z128, z1)


def _dense2_body(acc_ref, rs_ref, wout_ref, aos_ref, aod_ref,
                 h2_ref, f2s_ref, f2d_ref):
    h2 = jnp.zeros((BLK, NHID), jnp.float32)
    for p in range(NPAIR):
        for q in range(2):
            x1h = (acc_ref[p, :, q * NHID:(q + 1) * NHID]
                   / (rs_ref[p, :, q:q + 1] + 1e-16))
            x1h = jnp.where(x1h > 0.0, x1h, jnp.exp(x1h) - 1.0)
            h2 = h2 + jnp.dot(x1h, wout_ref[2 * p + q],
                              preferred_element_type=jnp.float32)
    h2_ref[...] = jnp.concatenate([h2, jnp.zeros((BLK, NHID), jnp.float32)],
                                  axis=1)
    f2s_ref[...] = jnp.dot(h2, aos_ref[...], preferred_element_type=jnp.float32)
    f2d_ref[...] = jnp.dot(h2, aod_ref[...], preferred_element_type=jnp.float32)


def _dense2(acc4, rs4, wout3, aos, aod):
    return pl.pallas_call(
        _dense2_body,
        grid=(N2 // BLK,),
        in_specs=[
            pl.BlockSpec((NPAIR, BLK, PW), lambda i: (0, i, 0)),
            pl.BlockSpec((NPAIR, BLK, 2), lambda i: (0, i, 0)),
            pl.BlockSpec((NHEADS, NHID, NHID), lambda i: (0, 0, 0)),
            pl.BlockSpec((NHID, 1), lambda i: (0, 0)),
            pl.BlockSpec((NHID, 1), lambda i: (0, 0)),
        ],
        out_specs=[
            pl.BlockSpec((BLK, PW), lambda i: (i, 0)),
            pl.BlockSpec((BLK, 1), lambda i: (i, 0)),
            pl.BlockSpec((BLK, 1), lambda i: (i, 0)),
        ],
        out_shape=[
            jax.ShapeDtypeStruct((N2, PW), jnp.float32),
            jax.ShapeDtypeStruct((N2, 1), jnp.float32),
            jax.ShapeDtypeStruct((N2, 1), jnp.float32),
        ],
    )(acc4, rs4, wout3, aos, aod)


def _edges2(bsrc, bdst, cnts, f2s_flat, f2d_flat, h2, z128, z1):
    @functools.partial(
        pl.kernel,
        out_type=[
            jax.ShapeDtypeStruct((2 * N2, PW), jnp.float32),
            jax.ShapeDtypeStruct((2 * N2,), jnp.float32),
        ],
        mesh=plsc.VectorSubcoreMesh(core_axis_name="c", subcore_axis_name="s"),
        scratch_types=[
            pltpu.VMEM_SHARED((N2, PW), jnp.float32),
            pltpu.VMEM_SHARED((N2,), jnp.float32),
            pltpu.VMEM((512,), jnp.int32),
            pltpu.VMEM((CB,), jnp.int32),
            pltpu.VMEM((CB,), jnp.int32),
            pltpu.VMEM((CB,), jnp.float32),
            pltpu.VMEM((CB,), jnp.float32),
            pltpu.VMEM((CB,), jnp.float32),
            pltpu.VMEM((CB, PW), jnp.float32),
            pltpu.SemaphoreType.DMA,
        ],
    )
    def k(bs_r, bd_r, cnt_r, fs_r, fd_r, h_r, z128_r, z1_r, acc_out, rs_out,
          acc_sh, rs_sh, cbuf, sidx, didx, fs_v, fd_v, w_v, rows_v, sem):
        c = lax.axis_index("c")
        s = lax.axis_index("s")
        pltpu.sync_copy(cnt_r, cbuf)
        s16 = jnp.broadcast_to(s, (16,))
        iota = lax.iota(jnp.int32, 16)
        pltpu.sync_copy(z128_r.at[pl.ds(s * NPT, NPT)],
                        acc_sh.at[pl.ds(s * NPT, NPT)])
        pltpu.sync_copy(z1_r.at[pl.ds(s * NPT, NPT)],
                        rs_sh.at[pl.ds(s * NPT, NPT)])
        plsc.subcore_barrier()

        def chunk_body(kk, seg_base):
            base = seg_base + kk * CB
            pltpu.sync_copy(bs_r.at[pl.ds(base, CB)], sidx)
            pltpu.sync_copy(bd_r.at[pl.ds(base, CB)], didx)
            rows_cp = pltpu.async_copy(h_r.at[didx], rows_v, sem)
            pltpu.sync_copy(fs_r.at[sidx], fs_v)
            pltpu.sync_copy(fd_r.at[didx], fd_v)
            for i in range(CB // 16):
                sl = pl.ds(i * 16, 16)
                e = fs_v[sl] + fd_v[sl]
                e = jnp.where(e >= 0.0, e, ALPHA * e)
                w_v[sl] = jnp.exp(-e)
            pltpu.sync_copy(w_v, rs_sh.at[sidx], add=True)
            rows_cp.wait()
            _scale_rows2(w_v, w_v, rows_v, CB // 16)
            pltpu.sync_copy(rows_v, acc_sh.at[sidx], add=True)
            return seg_base

        def seg_body(w, carry):
            w2 = c * 16 + w
            row = cbuf[pl.ds(pl.multiple_of(w2 * 16, 16), 16)]
            cnt = jnp.int32(0)
            for b in range(16):
                cnt = lax.select(s == b, row[b], cnt)
            nch = jnp.right_shift(cnt, 7)
            seg_base = w2 * WSEG + s * SEG
            lax.fori_loop(0, nch, chunk_body, seg_base)
            return carry
        lax.fori_loop(0, 16, seg_body, 0)

        plsc.subcore_barrier()
        pltpu.sync_copy(acc_sh.at[pl.ds(s * NPT, NPT)],
                        acc_out.at[pl.ds(c * N2 + s * NPT, NPT)])
        pltpu.sync_copy(rs_sh.at[pl.ds(s * NPT, NPT)],
                        rs_out.at[pl.ds(c * N2 + s * NPT, NPT)])

    return k(bsrc, bdst, cnts, f2s_flat, f2d_flat, h2, z128, z1)


def _merge_body(acc_ref, rs_ref, gat_ref):
    tot = acc_ref[0, :, :NHID] + acc_ref[1, :, :NHID]
    rs = rs_ref[0] + rs_ref[1] + 1e-16
    g = tot / rs
    g = jnp.where(g > 0.0, g, jnp.exp(g) - 1.0)
    gat_ref[...] = jnp.concatenate(
        [g, jnp.zeros((BLK, NHID), jnp.float32)], axis=1)


def _merge(acc2r, rs2r):
    return pl.pallas_call(
        _merge_body,
        grid=(N2 // BLK,),
        in_specs=[
            pl.BlockSpec((2, BLK, PW), lambda i: (0, i, 0)),
            pl.BlockSpec((2, BLK, 1), lambda i: (0, i, 0)),
        ],
        out_specs=pl.BlockSpec((BLK, PW), lambda i: (i, 0)),
        out_shape=jax.ShapeDtypeStruct((N2, PW), jnp.float32),
    )(acc2r, rs2r)


def _gather_fm(gat, type_index, nzi_flat, nzv_flat, zf):
    RPT = B // 32
    ZS = RPT * NFEAT
    HB = B // 2

    @functools.partial(
        pl.kernel,
        out_type=[
            jax.ShapeDtypeStruct((B, PW), jnp.float32),
            jax.ShapeDtypeStruct((B * NFEAT,), jnp.float32),
            jax.ShapeDtypeStruct((B * NFEAT,), jnp.float32),
        ],
        mesh=plsc.VectorSubcoreMesh(core_axis_name="c", subcore_axis_name="s"),
        scratch_types=[
            pltpu.VMEM_SHARED((HB * NFEAT,), jnp.float32),
            pltpu.VMEM_SHARED((HB * NFEAT,), jnp.float32),
            pltpu.VMEM((RPT,), jnp.int32),
            pltpu.VMEM((RPT, PW), jnp.float32),
            pltpu.VMEM((RPT * NNZ,), jnp.int32),
            pltpu.VMEM((RPT * NNZ,), jnp.float32),
            pltpu.VMEM((RPT * NNZ,), jnp.float32),
            pltpu.VMEM((RPT * NNZ,), jnp.int32),
        ],
    )
    def k(gat_r, ti_r, nzi_r, nzv_r, zf_r, g_out, s_out, s2_out,
          s_sh, s2_sh, tib, grows, nzib, nzvb, nzv2b, tidxb):
        c = lax.axis_index("c")
        s = lax.axis_index("s")
        row0 = c * HB + s * RPT
        pltpu.sync_copy(zf_r.at[pl.ds(s * ZS, ZS)], s_sh.at[pl.ds(s * ZS, ZS)])
        pltpu.sync_copy(zf_r.at[pl.ds(s * ZS, ZS)], s2_sh.at[pl.ds(s * ZS, ZS)])
        pltpu.sync_copy(ti_r.at[pl.ds(row0, RPT)], tib)
        pltpu.sync_copy(gat_r.at[tib], grows)
        pltpu.sync_copy(grows, g_out.at[pl.ds(row0, RPT)])
        pltpu.sync_copy(nzi_r.at[pl.ds(row0 * NNZ, RPT * NNZ)], nzib)
        pltpu.sync_copy(nzv_r.at[pl.ds(row0 * NNZ, RPT * NNZ)], nzvb)
        for r in range(RPT):
            sl = pl.ds(r * NNZ, 16)
            tidxb[sl] = nzib[sl] + (s * RPT + r) * NFEAT
            v = nzvb[sl]
            nzv2b[sl] = v * v
        plsc.subcore_barrier()
        pltpu.sync_copy(nzvb, s_sh.at[tidxb], add=True)
        pltpu.sync_copy(nzv2b, s2_sh.at[tidxb], add=True)
        plsc.subcore_barrier()
        pltpu.sync_copy(s_sh.at[pl.ds(s * ZS, ZS)],
                        s_out.at[pl.ds(c * HB * NFEAT + s * ZS, ZS)])
        pltpu.sync_copy(s2_sh.at[pl.ds(s * ZS, ZS)],
                        s2_out.at[pl.ds(c * HB * NFEAT + s * ZS, ZS)])

    return k(gat, type_index, nzi_flat, nzv_flat, zf)


def _fusion_body(g_ref, s_ref, s2_ref, vfm_ref, wa_ref, ba_ref, ha_ref,
                 wf_ref, bf_ref, out_ref):
    g = g_ref[:, :NHID]
    vfm = vfm_ref[...]
    sv = jnp.dot(s_ref[...], vfm, preferred_element_type=jnp.float32)
    s2v = jnp.dot(s2_ref[...], vfm * vfm, preferred_element_type=jnp.float32)
    fm = 0.5 * (sv * sv - s2v)
    ba = ba_ref[...]
    ha = ha_ref[...]
    wa = wa_ref[...]
    ag = jnp.dot(jnp.tanh(jnp.dot(g, wa, preferred_element_type=jnp.float32) + ba),
                 ha, preferred_element_type=jnp.float32)
    af = jnp.dot(jnp.tanh(jnp.dot(fm, wa, preferred_element_type=jnp.float32) + ba),
                 ha, preferred_element_type=jnp.float32)
    m = jnp.maximum(ag, af)
    e0 = jnp.exp(ag - m)
    e1 = jnp.exp(af - m)
    tot = e0 + e1
    a0 = e0 / tot
    a1 = e1 / tot
    wf = wf_ref[...]
    out = (jnp.dot(g * a0, wf[:NHID], preferred_element_type=jnp.float32)
           + jnp.dot(fm * a1, wf[NHID:], preferred_element_type=jnp.float32)
           + bf_ref[...])
    mx = jnp.max(out, axis=1, keepdims=True)
    sh = out - mx
    lse = jnp.log(jnp.sum(jnp.exp(sh), axis=1, keepdims=True))
    out_ref[...] = sh - lse


def _fusion(g, s, s2, vfm, wa, ba, ha, wf, bf):
    FB = 512
    return pl.pallas_call(
        _fusion_body,
        grid=(B // FB,),
        in_specs=[
            pl.BlockSpec((FB, PW), lambda i: (i, 0)),
            pl.BlockSpec((FB, NFEAT), lambda i: (i, 0)),
            pl.BlockSpec((FB, NFEAT), lambda i: (i, 0)),
            pl.BlockSpec((NFEAT, NHID), lambda i: (0, 0)),
            pl.BlockSpec((NHID, NHID), lambda i: (0, 0)),
            pl.BlockSpec((1, NHID), lambda i: (0, 0)),
            pl.BlockSpec((NHID, 1), lambda i: (0, 0)),
            pl.BlockSpec((2 * NHID, NCLASS), lambda i: (0, 0)),
            pl.BlockSpec((1, NCLASS), lambda i: (0, 0)),
        ],
        out_specs=pl.BlockSpec((FB, NCLASS), lambda i: (i, 0)),
        out_shape=jax.ShapeDtypeStruct((B, NCLASS), jnp.float32),
    )(g, s, s2, vfm, wa, ba, ha, wf, bf)


def kernel(x, adj, type_index, non_zero_index, non_zero_value, W_heads, a_heads,
           W_out, a_out, V_fm, W_attn, b_attn, h_attn, W_final, b_final):
    x_p = jnp.zeros((N2, NFEAT), jnp.float32).at[:N_NODES].set(x)
    wpair = (W_heads.reshape(NPAIR, 2, NFEAT, NHID)
             .transpose(0, 2, 1, 3).reshape(NPAIR, NFEAT, PW))
    a2 = a_heads.reshape(NHEADS, 2 * NHID)
    asrc = a2[:, :NHID].reshape(NPAIR, 2, NHID)
    adst = a2[:, NHID:].reshape(NPAIR, 2, NHID)
    eye2 = jnp.eye(2, dtype=jnp.float32)
    asrc_bd = (asrc[:, :, :, None] * eye2[None, :, None, :]).reshape(NPAIR, PW, 2)
    adst_bd = (adst[:, :, :, None] * eye2[None, :, None, :]).reshape(NPAIR, PW, 2)

    h3, fs3, fd3 = _dense1(x_p, wpair, asrc_bd, adst_bd)
    h_flat = h3.reshape(NPAIR * N2, PW)
    fs_flat = fs3.reshape(NPAIR * N2 * 2)
    fd_flat = fd3.reshape(NPAIR * N2 * 2)

    zbig = jnp.zeros((N2 * PW,), jnp.float32)
    z128 = zbig.reshape(N2, PW)
    z1 = zbig[:2 * N2]
    zf = zbig[:(B // 2) * NFEAT]

    src_e = adj[0]
    dst_e = adj[1]
    bsrc, bdst, cnts = _binedges(src_e, dst_e)
    acc_flat, rs_flat = _edges1(bsrc, bdst, cnts, fs_flat, fd_flat,
                                h_flat, z128, z1)
    acc4 = acc_flat.reshape(NPAIR, N2, PW)
    rs4 = rs_flat.reshape(NPAIR, N2, 2)

    wout3 = W_out.reshape(NHEADS, NHID, NHID)
    ao = a_out.reshape(2 * NHID)
    aos = ao[:NHID].reshape(NHID, 1)
    aod = ao[NHID:].reshape(NHID, 1)
    h2, f2s, f2d = _dense2(acc4, rs4, wout3, aos, aod)

    acc2_flat, rs2_flat = _edges2(bsrc, bdst, cnts, f2s.reshape(N2),
                                  f2d.reshape(N2), h2, z128, z1[:N2])
    gat = _merge(acc2_flat.reshape(2, N2, PW), rs2_flat.reshape(2, N2, 1))

    g, s_flat, s2_flat = _gather_fm(gat, type_index,
                                    non_zero_index.reshape(B * NNZ),
                                    non_zero_value.reshape(B * NNZ), zf)

    return _fusion(g, s_flat.reshape(B, NFEAT), s2_flat.reshape(B, NFEAT),
                   V_fm, W_attn, b_attn.reshape(1, NHID),
                   h_attn.reshape(NHID, 1), W_final,
                   b_final.reshape(1, NCLASS))
